# Initial kernel scaffold; baseline (speedup 1.0000x reference)
import jax
import jax.numpy as jnp
from jax.experimental import pallas as pl


def kernel(x, edge_index, edge_attr, batch, W1s, b1s, gammas, betas, bn_mean, bn_var, W2s, b2s, Wes, bes, lin1_W, lin1_b, lin2_W, lin2_b):
    raise NotImplementedError("write your pallas kernel here")


# speedup vs baseline: 2.7351x; 2.7351x over previous
import functools

import jax
import jax.numpy as jnp
from jax import lax
from jax.experimental import pallas as pl
from jax.experimental.pallas import tpu as pltpu
from jax.experimental.pallas import tpu_sc as plsc

N = 10000
E = 320000
D = 128
DE = 16
H = 128
L = 3
G = 128

NC = 2
NS = 16
NW = NC * NS
EPW = E // NW
K = 80
NB = EPW // K
RPT = N // NS
CB = 25
NCH = NB // CB


def _sc_edge_body(h_hbm, ea_hbm, src_hbm, dst_hbm, out_hbm,
                  src_v, dst_v, rows_v, ea_v, acc_sh, sem_g, sem_e):
    cid = lax.axis_index("c")
    sid = lax.axis_index("s")
    wid = cid * NS + sid

    zeros16 = jnp.zeros((16,), jnp.float32)

    def _zero_row(i, _):
        for j in range(8):
            ea_v[i, pl.ds(j * 16, 16)] = zeros16
        return 0

    lax.fori_loop(0, K, _zero_row, 0)
    for r in range(RPT // K):
        pltpu.sync_copy(ea_v, acc_sh.at[pl.ds(sid * RPT + r * K, K)])
    pltpu.sync_copy(ea_v.at[pl.ds(0, RPT % K)],
                    acc_sh.at[pl.ds(sid * RPT + (RPT // K) * K, RPT % K)])
    plsc.subcore_barrier()

    def _block(b, _):
        pltpu.sync_copy(src_hbm.at[wid, b], src_v)
        pltpu.sync_copy(dst_hbm.at[wid, b], dst_v)
        gcp = pltpu.async_copy(h_hbm.at[src_v], rows_v, sem_g)
        ecp = pltpu.async_copy(ea_hbm.at[wid, b], ea_v, sem_e)
        gcp.wait()
        ecp.wait()

        def _row(i, _):
            for j in range(8):
                sl = pl.ds(j * 16, 16)
                rows_v[i, sl] = jnp.maximum(rows_v[i, sl] + ea_v[i, sl], 0.0)
            return 0

        lax.fori_loop(0, K, _row, 0)
        pltpu.sync_copy(rows_v, acc_sh.at[dst_v], add=True)
        return 0

    lax.fori_loop(0, NB, _block, 0)
    plsc.subcore_barrier()
    pltpu.sync_copy(acc_sh.at[pl.ds(sid * RPT, RPT)], out_hbm.at[wid])


@functools.cache
def _sc_edge_kernel():
    return pl.kernel(
        _sc_edge_body,
        out_type=jax.ShapeDtypeStruct((NW, RPT, D), jnp.float32),
        mesh=plsc.VectorSubcoreMesh(core_axis_name="c", subcore_axis_name="s",
                                    num_cores=NC, num_subcores=NS),
        scratch_types=[
            pltpu.VMEM((K,), jnp.int32),
            pltpu.VMEM((K,), jnp.int32),
            pltpu.VMEM((K, D), jnp.float32),
            pltpu.VMEM((K, D), jnp.float32),
            pltpu.VMEM_SHARED((N, D), jnp.float32),
            pltpu.SemaphoreType.DMA,
            pltpu.SemaphoreType.DMA,
        ],
    )


EB = 4000


def _ea_body(ea_ref, w_ref, b_ref, out_ref):
    out_ref[...] = (
        jnp.dot(ea_ref[...], w_ref[...], preferred_element_type=jnp.float32)
        + b_ref[...]
    )


def _ea_call(edge_attr, w, b):
    return pl.pallas_call(
        _ea_body,
        grid=(E // EB,),
        in_specs=[
            pl.BlockSpec((EB, DE), lambda i: (i, 0)),
            pl.BlockSpec((DE, D), lambda i: (0, 0)),
            pl.BlockSpec((1, D), lambda i: (0, 0)),
        ],
        out_specs=pl.BlockSpec((EB, D), lambda i: (i, 0)),
        out_shape=jax.ShapeDtypeStruct((E, D), jnp.float32),
    )(edge_attr, w, b)


NBK = 1000


def _mlp_body(h_ref, a0_ref, a1_ref, b_ref, w1_ref, b1_ref, w2_ref, b2_ref,
              hout_ref, pool_ref):
    pid = pl.program_id(0)
    z = h_ref[...] + a0_ref[...] + a1_ref[...]
    z = jnp.dot(z, w1_ref[...], preferred_element_type=jnp.float32) + b1_ref[...]
    z = jnp.maximum(z, 0.0)
    z = jnp.dot(z, w2_ref[...], preferred_element_type=jnp.float32) + b2_ref[...]
    h2 = jnp.maximum(z, 0.0)
    hout_ref[...] = h2
    seg = b_ref[0]
    oh_t = (lax.broadcasted_iota(jnp.int32, (G, NBK), 0) == seg).astype(jnp.float32)
    contrib = jnp.dot(oh_t, h2, preferred_element_type=jnp.float32)

    @pl.when(pid == 0)
    def _init():
        pool_ref[...] = jnp.zeros_like(pool_ref)

    pool_ref[...] += contrib


def _mlp_call(h, a0, a1, batch3, w1, b1, w2, b2):
    return pl.pallas_call(
        _mlp_body,
        grid=(N // NBK,),
        in_specs=[
            pl.BlockSpec((NBK, D), lambda i: (i, 0)),
            pl.BlockSpec((NBK, D), lambda i: (i, 0)),
            pl.BlockSpec((NBK, D), lambda i: (i, 0)),
            pl.BlockSpec((1, 1, NBK), lambda i: (i, 0, 0)),
            pl.BlockSpec((D, H), lambda i: (0, 0)),
            pl.BlockSpec((1, H), lambda i: (0, 0)),
            pl.BlockSpec((H, H), lambda i: (0, 0)),
            pl.BlockSpec((1, H), lambda i: (0, 0)),
        ],
        out_specs=[
            pl.BlockSpec((NBK, H), lambda i: (i, 0)),
            pl.BlockSpec((G, H), lambda i: (0, 0)),
        ],
        out_shape=[
            jax.ShapeDtypeStruct((N, H), jnp.float32),
            jax.ShapeDtypeStruct((G, H), jnp.float32),
        ],
    )(h, a0, a1, batch3, w1, b1, w2, b2)


def _head_body(p0_ref, p1_ref, p2_ref, w1_ref, b1_ref, w2_ref, b2_ref, out_ref):
    w1 = w1_ref[...]
    t = (
        jnp.dot(p0_ref[...], w1[0:H, :], preferred_element_type=jnp.float32)
        + jnp.dot(p1_ref[...], w1[H:2 * H, :], preferred_element_type=jnp.float32)
        + jnp.dot(p2_ref[...], w1[2 * H:3 * H, :], preferred_element_type=jnp.float32)
        + b1_ref[...]
    )
    t = jnp.maximum(t, 0.0)
    out_ref[...] = (
        jnp.dot(t, w2_ref[...], preferred_element_type=jnp.float32) + b2_ref[...]
    )


def _head_call(p0, p1, p2, w1, b1, w2, b2):
    return pl.pallas_call(
        _head_body,
        out_shape=jax.ShapeDtypeStruct((G, 1), jnp.float32),
    )(p0, p1, p2, w1, b1, w2, b2)


def kernel(x, edge_index, edge_attr, batch, W1s, b1s, gammas, betas, bn_mean,
           bn_var, W2s, b2s, Wes, bes, lin1_W, lin1_b, lin2_W, lin2_b):
    src3 = edge_index[0].reshape(NW, NB, K)
    dst3 = edge_index[1].reshape(NW, NB, K)
    batch3 = batch.reshape(N // NBK, 1, NBK)

    scale = gammas / jnp.sqrt(bn_var + 1e-5)
    w1e = W1s * scale[:, None, :]
    b1e = (b1s - bn_mean) * scale + betas

    h = x
    pools = []
    for l in range(L):
        ea = _ea_call(edge_attr, Wes[l], bes[l].reshape(1, D))
        parts = _sc_edge_kernel()(h, ea.reshape(NW, NB, K, D), src3,
                                  dst3).reshape(NC, N, D)
        h, pool = _mlp_call(h, parts[0], parts[1], batch3,
                            w1e[l], b1e[l].reshape(1, H),
                            W2s[l], b2s[l].reshape(1, H))
        pools.append(pool)

    return _head_call(pools[0], pools[1], pools[2],
                      lin1_W, lin1_b.reshape(1, H * L),
                      lin2_W, lin2_b.reshape(1, 1))


# speedup vs baseline: 3.8573x; 1.4103x over previous
import functools

import jax
import jax.numpy as jnp
from jax import lax
from jax.experimental import pallas as pl
from jax.experimental.pallas import tpu as pltpu
from jax.experimental.pallas import tpu_sc as plsc

N = 10000
E = 320000
D = 128
DE = 16
H = 128
L = 3
G = 128

NC = 2
NS = 16
NW = NC * NS
EPW = E // NW
K = 40
NB = EPW // K
RPT = N // NS


def _sc_edge_body(h_hbm, ea_hbm, src_hbm, dst_hbm, out_hbm,
                  src0, src1, dst0, dst1, rows0, rows1, ea0, ea1, acc_sh,
                  sg0, sg1, se0, se1, sx0, sx1, sy0, sy1, ss0, ss1):
    cid = lax.axis_index("c")
    sid = lax.axis_index("s")
    wid = cid * NS + sid
    src_v = (src0, src1)
    dst_v = (dst0, dst1)
    rows_v = (rows0, rows1)
    ea_v = (ea0, ea1)
    sem_g = (sg0, sg1)
    sem_e = (se0, se1)
    sem_x = (sx0, sx1)
    sem_y = (sy0, sy1)
    sem_s = (ss0, ss1)

    zeros16 = jnp.zeros((16,), jnp.float32)

    def _zero_row(i, _):
        for j in range(8):
            ea0[i, pl.ds(j * 16, 16)] = zeros16
        return 0

    lax.fori_loop(0, K, _zero_row, 0)
    for r in range(RPT // K):
        pltpu.sync_copy(ea0, acc_sh.at[pl.ds(sid * RPT + r * K, K)])
    pltpu.sync_copy(ea0.at[pl.ds(0, RPT % K)],
                    acc_sh.at[pl.ds(sid * RPT + (RPT // K) * K, RPT % K)])
    plsc.subcore_barrier()

    def issue_gather(s):
        pltpu.async_copy(h_hbm.at[src_v[s]], rows_v[s], sem_g[s])

    def wait_gather(s):
        pltpu.make_async_copy(h_hbm.at[src_v[s]], rows_v[s], sem_g[s]).wait()

    def issue_ea(b, s):
        pltpu.async_copy(ea_hbm.at[wid, b], ea_v[s], sem_e[s])

    def wait_ea(b, s):
        pltpu.make_async_copy(ea_hbm.at[wid, b], ea_v[s], sem_e[s]).wait()

    def issue_src(b, s):
        pltpu.async_copy(src_hbm.at[wid, b], src_v[s], sem_x[s])

    def wait_src(b, s):
        pltpu.make_async_copy(src_hbm.at[wid, b], src_v[s], sem_x[s]).wait()

    def issue_dst(b, s):
        pltpu.async_copy(dst_hbm.at[wid, b], dst_v[s], sem_y[s])

    def wait_dst(b, s):
        pltpu.make_async_copy(dst_hbm.at[wid, b], dst_v[s], sem_y[s]).wait()

    def issue_scatter(s):
        pltpu.async_copy(rows_v[s], acc_sh.at[dst_v[s]], sem_s[s], add=True)

    def wait_scatter(s):
        pltpu.make_async_copy(rows_v[s], acc_sh.at[dst_v[s]],
                              sem_s[s]).wait()

    pltpu.sync_copy(src_hbm.at[wid, 0], src_v[0])
    issue_gather(0)
    issue_ea(0, 0)
    issue_dst(0, 0)
    issue_src(1, 1)

    def _macro(m, _):
        for s in (0, 1):
            b = 2 * m + s
            q = 1 - s
            wait_gather(s)
            wait_ea(b, s)

            @pl.when(b + 2 < NB)
            def _():
                issue_src(b + 2, s)

            @pl.when(b > 0)
            def _():
                wait_scatter(q)

            @pl.when(b + 1 < NB)
            def _():
                wait_src(b + 1, q)
                issue_gather(q)
                issue_ea(b + 1, q)
                issue_dst(b + 1, q)

            def _row(i, _):
                for j in range(8):
                    sl = pl.ds(j * 16, 16)
                    rows_v[s][i, sl] = jnp.maximum(
                        rows_v[s][i, sl] + ea_v[s][i, sl], 0.0)
                return 0

            lax.fori_loop(0, K, _row, 0)

            wait_dst(b, s)
            issue_scatter(s)
        return 0

    lax.fori_loop(0, NB // 2, _macro, 0)
    wait_scatter(1)
    plsc.subcore_barrier()
    pltpu.sync_copy(acc_sh.at[pl.ds(sid * RPT, RPT)], out_hbm.at[wid])


@functools.cache
def _sc_edge_kernel():
    return pl.kernel(
        _sc_edge_body,
        out_type=jax.ShapeDtypeStruct((NW, RPT, D), jnp.float32),
        mesh=plsc.VectorSubcoreMesh(core_axis_name="c", subcore_axis_name="s",
                                    num_cores=NC, num_subcores=NS),
        scratch_types=(
            [pltpu.VMEM((K,), jnp.int32)] * 4
            + [pltpu.VMEM((K, D), jnp.float32)] * 4
            + [pltpu.VMEM_SHARED((N, D), jnp.float32)]
            + [pltpu.SemaphoreType.DMA] * 10
        ),
    )


EB = 4000


def _ea_body(ea_ref, w_ref, b_ref, out_ref):
    out_ref[...] = (
        jnp.dot(ea_ref[...], w_ref[...], preferred_element_type=jnp.float32)
        + b_ref[...]
    )


def _ea_call(edge_attr, w, b):
    return pl.pallas_call(
        _ea_body,
        grid=(E // EB,),
        in_specs=[
            pl.BlockSpec((EB, DE), lambda i: (i, 0)),
            pl.BlockSpec((DE, D), lambda i: (0, 0)),
            pl.BlockSpec((1, D), lambda i: (0, 0)),
        ],
        out_specs=pl.BlockSpec((EB, D), lambda i: (i, 0)),
        out_shape=jax.ShapeDtypeStruct((E, D), jnp.float32),
    )(edge_attr, w, b)


NBK = 1000


def _mlp_body(h_ref, a0_ref, a1_ref, b_ref, w1_ref, b1_ref, w2_ref, b2_ref,
              hout_ref, pool_ref):
    pid = pl.program_id(0)
    z = h_ref[...] + a0_ref[...] + a1_ref[...]
    z = jnp.dot(z, w1_ref[...], preferred_element_type=jnp.float32) + b1_ref[...]
    z = jnp.maximum(z, 0.0)
    z = jnp.dot(z, w2_ref[...], preferred_element_type=jnp.float32) + b2_ref[...]
    h2 = jnp.maximum(z, 0.0)
    hout_ref[...] = h2
    seg = b_ref[0]
    oh_t = (lax.broadcasted_iota(jnp.int32, (G, NBK), 0) == seg).astype(jnp.float32)
    contrib = jnp.dot(oh_t, h2, preferred_element_type=jnp.float32)

    @pl.when(pid == 0)
    def _init():
        pool_ref[...] = jnp.zeros_like(pool_ref)

    pool_ref[...] += contrib


def _mlp_call(h, a0, a1, batch3, w1, b1, w2, b2):
    return pl.pallas_call(
        _mlp_body,
        grid=(N // NBK,),
        in_specs=[
            pl.BlockSpec((NBK, D), lambda i: (i, 0)),
            pl.BlockSpec((NBK, D), lambda i: (i, 0)),
            pl.BlockSpec((NBK, D), lambda i: (i, 0)),
            pl.BlockSpec((1, 1, NBK), lambda i: (i, 0, 0)),
            pl.BlockSpec((D, H), lambda i: (0, 0)),
            pl.BlockSpec((1, H), lambda i: (0, 0)),
            pl.BlockSpec((H, H), lambda i: (0, 0)),
            pl.BlockSpec((1, H), lambda i: (0, 0)),
        ],
        out_specs=[
            pl.BlockSpec((NBK, H), lambda i: (i, 0)),
            pl.BlockSpec((G, H), lambda i: (0, 0)),
        ],
        out_shape=[
            jax.ShapeDtypeStruct((N, H), jnp.float32),
            jax.ShapeDtypeStruct((G, H), jnp.float32),
        ],
    )(h, a0, a1, batch3, w1, b1, w2, b2)


def _head_body(p0_ref, p1_ref, p2_ref, w1_ref, b1_ref, w2_ref, b2_ref, out_ref):
    w1 = w1_ref[...]
    t = (
        jnp.dot(p0_ref[...], w1[0:H, :], preferred_element_type=jnp.float32)
        + jnp.dot(p1_ref[...], w1[H:2 * H, :], preferred_element_type=jnp.float32)
        + jnp.dot(p2_ref[...], w1[2 * H:3 * H, :], preferred_element_type=jnp.float32)
        + b1_ref[...]
    )
    t = jnp.maximum(t, 0.0)
    out_ref[...] = (
        jnp.dot(t, w2_ref[...], preferred_element_type=jnp.float32) + b2_ref[...]
    )


def _head_call(p0, p1, p2, w1, b1, w2, b2):
    return pl.pallas_call(
        _head_body,
        out_shape=jax.ShapeDtypeStruct((G, 1), jnp.float32),
    )(p0, p1, p2, w1, b1, w2, b2)


def kernel(x, edge_index, edge_attr, batch, W1s, b1s, gammas, betas, bn_mean,
           bn_var, W2s, b2s, Wes, bes, lin1_W, lin1_b, lin2_W, lin2_b):
    src3 = edge_index[0].reshape(NW, NB, K)
    dst3 = edge_index[1].reshape(NW, NB, K)
    batch3 = batch.reshape(N // NBK, 1, NBK)

    scale = gammas / jnp.sqrt(bn_var + 1e-5)
    w1e = W1s * scale[:, None, :]
    b1e = (b1s - bn_mean) * scale + betas

    h = x
    pools = []
    for l in range(L):
        ea = _ea_call(edge_attr, Wes[l], bes[l].reshape(1, D))
        parts = _sc_edge_kernel()(h, ea.reshape(NW, NB, K, D), src3,
                                  dst3).reshape(NC, N, D)
        h, pool = _mlp_call(h, parts[0], parts[1], batch3,
                            w1e[l], b1e[l].reshape(1, H),
                            W2s[l], b2s[l].reshape(1, H))
        pools.append(pool)

    return _head_call(pools[0], pools[1], pools[2],
                      lin1_W, lin1_b.reshape(1, H * L),
                      lin2_W, lin2_b.reshape(1, 1))
